# Initial kernel scaffold; baseline (speedup 1.0000x reference)
import jax
import jax.numpy as jnp
from jax.experimental import pallas as pl


def kernel(edge_index, pos_edge_index, neg_edge_index, user_emb, item_emb):
    raise NotImplementedError("write your pallas kernel here")


# speedup vs baseline: 3.0109x; 3.0109x over previous
import functools

import jax
import jax.numpy as jnp
from jax import lax
from jax.experimental import pallas as pl
from jax.experimental.pallas import tpu as pltpu
from jax.experimental.pallas import tpu_sc as plsc

U = 5000
I = 5000
E = 320000
EP = 100000
D = 128
L = 3

NC = 2
NS = 16
NW = NC * NS
LANES = 16

KE = 128
KP = 128
R = 5120
RPT = R // NS

EPAD = 327680
CH_E = EPAD // (NW * KE)
PPAD = 102400
CH_P = PPAD // (NW * KP)

DEGW = 16

_mesh = plsc.VectorSubcoreMesh(core_axis_name="c", subcore_axis_name="s")
_cparams = pltpu.CompilerParams(needs_layout_passes=False)


def _zero_fill(ref, nrows, ncols):
  z = jnp.zeros((LANES,), jnp.float32)
  for r in range(nrows):
    for c in range(ncols // LANES):
      ref[r, pl.ds(c * LANES, LANES)] = z


def _fill_const(ref, nrows, ncols, val):
  v = jnp.full((LANES,), val, jnp.float32)
  for r in range(nrows):
    for c in range(ncols // LANES):
      ref[r, pl.ds(c * LANES, LANES)] = v


@functools.partial(
    pl.kernel,
    out_type=(
        jax.ShapeDtypeStruct((NW, R), jnp.float32),
        jax.ShapeDtypeStruct((NW, R), jnp.float32),
    ),
    mesh=_mesh,
    compiler_params=_cparams,
    scratch_types=[
        pltpu.VMEM((LANES, R), jnp.float32),
        pltpu.VMEM((R,), jnp.float32),
        pltpu.VMEM((KE,), jnp.int32),
    ],
)
def _k_deg(src_hbm, dst_hbm, outu_hbm, outi_hbm, tab, robuf, sbuf):
  c = lax.axis_index("c")
  s = lax.axis_index("s")
  wid = c * NS + s
  lane = lax.iota(jnp.int32, LANES)
  ones = jnp.ones((LANES,), jnp.float32)
  zeros = jnp.zeros((LANES,), jnp.float32)

  def ztab(r, _):
    for l in range(LANES):
      tab[l, pl.ds(r * LANES, LANES)] = zeros
    return 0

  def count(idx_hbm):
    def body(j, _):
      pltpu.sync_copy(idx_hbm.at[wid, j], sbuf)
      for t in range(KE // LANES):
        idxv = sbuf[pl.ds(t * LANES, LANES)]
        plsc.addupdate_scatter(tab, [lane, idxv], ones)
      return 0
    lax.fori_loop(0, CH_E, body, 0)

  def reduce(r, _):
    acc = tab[0, pl.ds(r * LANES, LANES)]
    for l in range(1, LANES):
      acc = acc + tab[l, pl.ds(r * LANES, LANES)]
    robuf[pl.ds(r * LANES, LANES)] = acc
    return 0

  lax.fori_loop(0, R // LANES, ztab, 0)
  count(src_hbm)
  lax.fori_loop(0, R // LANES, reduce, 0)
  pltpu.sync_copy(robuf, outu_hbm.at[wid])

  lax.fori_loop(0, R // LANES, ztab, 0)
  count(dst_hbm)
  lax.fori_loop(0, R // LANES, reduce, 0)
  pltpu.sync_copy(robuf, outi_hbm.at[wid])


@functools.partial(
    pl.kernel,
    out_type=(
        jax.ShapeDtypeStruct((NC, R, D), jnp.float32),
        jax.ShapeDtypeStruct((NC, R, D), jnp.float32),
    ),
    mesh=_mesh,
    compiler_params=_cparams,
    scratch_types=[
        pltpu.VMEM_SHARED((R, D), jnp.float32),
        pltpu.VMEM_SHARED((R, D), jnp.float32),
        pltpu.VMEM((KE,), jnp.int32),
        pltpu.VMEM((KE,), jnp.int32),
        pltpu.VMEM((KE, D), jnp.float32),
    ],
)
def _k_layer(src_hbm, dst_hbm, pu_hbm, pi_hbm, outu_hbm, outi_hbm,
             accu_sh, acci_sh, sbuf, dbuf, rows_a):
  c = lax.axis_index("c")
  s = lax.axis_index("s")
  wid = c * NS + s
  _zero_fill(rows_a, KE, D)
  for base, w in ((0, KE), (KE, KE), (2 * KE, RPT - 2 * KE)):
    pltpu.sync_copy(rows_a.at[pl.ds(0, w)],
                    accu_sh.at[pl.ds(s * RPT + base, w)])
    pltpu.sync_copy(rows_a.at[pl.ds(0, w)],
                    acci_sh.at[pl.ds(s * RPT + base, w)])
  plsc.subcore_barrier()

  def body(j, _):
    pltpu.sync_copy(src_hbm.at[wid, j], sbuf)
    pltpu.sync_copy(dst_hbm.at[wid, j], dbuf)
    pltpu.sync_copy(pu_hbm.at[sbuf], rows_a)
    pltpu.sync_copy(rows_a, acci_sh.at[dbuf], add=True)
    pltpu.sync_copy(pi_hbm.at[dbuf], rows_a)
    pltpu.sync_copy(rows_a, accu_sh.at[sbuf], add=True)
    return 0

  lax.fori_loop(0, CH_E, body, 0)
  plsc.subcore_barrier()

  for base, w in ((0, KE), (KE, KE), (2 * KE, RPT - 2 * KE)):
    rb = pl.ds(s * RPT + base, w)
    pltpu.sync_copy(accu_sh.at[rb], rows_a.at[pl.ds(0, w)])
    pltpu.sync_copy(rows_a.at[pl.ds(0, w)], outu_hbm.at[c, rb])
    pltpu.sync_copy(acci_sh.at[rb], rows_a.at[pl.ds(0, w)])
    pltpu.sync_copy(rows_a.at[pl.ds(0, w)], outi_hbm.at[c, rb])


def _dot_chunk(rows_a, rows_b, out_vm, j):
  lane = lax.iota(jnp.int32, LANES)
  for g in range(KP // LANES):
    rowi = lane + g * LANES
    tot = jnp.zeros((LANES,), jnp.float32)
    for d in range(D):
      col = jnp.full((LANES,), d, jnp.int32)
      tot = tot + (plsc.load_gather(rows_a, [rowi, col])
                   * plsc.load_gather(rows_b, [rowi, col]))
    out_vm[j, pl.ds(g * LANES, LANES)] = tot


@functools.partial(
    pl.kernel,
    out_type=(
        jax.ShapeDtypeStruct((NW, CH_P, KP), jnp.float32),
        jax.ShapeDtypeStruct((NW, CH_P, KP), jnp.float32),
    ),
    mesh=_mesh,
    compiler_params=_cparams,
    scratch_types=[
        pltpu.VMEM((KP,), jnp.int32),
        pltpu.VMEM((KP,), jnp.int32),
        pltpu.VMEM((KP, D), jnp.float32),
        pltpu.VMEM((KP, D), jnp.float32),
        pltpu.VMEM((CH_P, KP), jnp.float32),
        pltpu.VMEM((CH_P, KP), jnp.float32),
        pltpu.SemaphoreType.DMA,
        pltpu.SemaphoreType.DMA,
    ],
)
def _k_score(ru_hbm, ri_hbm, pu_hbm, pi_hbm, nu_hbm, ni_hbm,
             outp_hbm, outn_hbm,
             abuf, bbuf, rows_a, rows_b,
             outp_vm, outn_vm, sem_a, sem_b):
  c = lax.axis_index("c")
  s = lax.axis_index("s")
  wid = c * NS + s

  def pbody(j, _):
    pltpu.sync_copy(pu_hbm.at[wid, j], abuf)
    pltpu.sync_copy(pi_hbm.at[wid, j], bbuf)
    ga = pltpu.async_copy(ru_hbm.at[abuf], rows_a, sem_a)
    gb = pltpu.async_copy(ri_hbm.at[bbuf], rows_b, sem_b)
    ga.wait()
    gb.wait()
    _dot_chunk(rows_a, rows_b, outp_vm, j)
    return 0

  def nbody(j, _):
    pltpu.sync_copy(nu_hbm.at[wid, j], abuf)
    pltpu.sync_copy(ni_hbm.at[wid, j], bbuf)
    ga = pltpu.async_copy(ru_hbm.at[abuf], rows_a, sem_a)
    gb = pltpu.async_copy(ri_hbm.at[bbuf], rows_b, sem_b)
    ga.wait()
    gb.wait()
    _dot_chunk(rows_a, rows_b, outn_vm, j)
    return 0

  lax.fori_loop(0, CH_P, pbody, 0)
  lax.fori_loop(0, CH_P, nbody, 0)
  pltpu.sync_copy(outp_vm, outp_hbm.at[wid])
  pltpu.sync_copy(outn_vm, outn_hbm.at[wid])


def _pad_reshape_idx(a, total, fill, ch, k):
  pad = total - a.shape[0]
  a = jnp.concatenate([a, jnp.full((pad,), fill, jnp.int32)])
  return a.reshape(NW, ch, k)


def _pad_rows(m):
  return jnp.concatenate(
      [m, jnp.zeros((R - m.shape[0], m.shape[1]), m.dtype)], axis=0)


def kernel(edge_index, pos_edge_index, neg_edge_index, user_emb, item_emb):
  src = edge_index[0]
  dst = edge_index[1]
  src3 = _pad_reshape_idx(src, EPAD, U, CH_E, KE)
  dst3 = _pad_reshape_idx(dst, EPAD, I, CH_E, KE)

  du, di = _k_deg(src3, dst3)
  deg_u = jnp.sum(du, axis=0)[:U]
  deg_i = jnp.sum(di, axis=0)[:I]
  inv_su = lax.rsqrt(jnp.maximum(deg_u, 1.0))
  inv_si = lax.rsqrt(jnp.maximum(deg_i, 1.0))

  h_u, h_i = user_emb, item_emb
  res_u, res_i = user_emb, item_emb
  for l in range(L):
    pu = _pad_rows(h_u * inv_su[:, None])
    pi = _pad_rows(h_i * inv_si[:, None])
    agg_u, agg_i = _k_layer(src3, dst3, pu, pi)
    h_u = (agg_u[0] + agg_u[1])[:U] * inv_su[:, None]
    h_i = (agg_i[0] + agg_i[1])[:I] * inv_si[:, None]
    res_u = res_u + h_u * (1.0 / (l + 2))
    res_i = res_i + h_i * (1.0 / (l + 2))

  pu3 = _pad_reshape_idx(pos_edge_index[0], PPAD, 0, CH_P, KP)
  pi3 = _pad_reshape_idx(pos_edge_index[1], PPAD, 0, CH_P, KP)
  nu3 = _pad_reshape_idx(neg_edge_index[0], PPAD, 0, CH_P, KP)
  ni3 = _pad_reshape_idx(neg_edge_index[1], PPAD, 0, CH_P, KP)
  outp, outn = _k_score(res_u, res_i, pu3, pi3, nu3, ni3)
  pos_score = outp.reshape(-1)[:EP, None]
  neg_score = outn.reshape(-1)[:EP, None]
  return (pos_score, neg_score)


# speedup vs baseline: 4.7742x; 1.5856x over previous
import functools

import jax
import jax.numpy as jnp
from jax import lax
from jax.experimental import pallas as pl
from jax.experimental.pallas import tpu as pltpu
from jax.experimental.pallas import tpu_sc as plsc

U = 5000
I = 5000
E = 320000
EP = 100000
D = 128
L = 3

NC = 2
NS = 16
NW = NC * NS
LANES = 16

KE = 80
KP = 128
R = 5120
RPT = R // NS

EPAD = 327680
CH_E = EPAD // (NW * KE)
PPAD = 102400
CH_P = PPAD // (NW * KP)
CHS = 2 * CH_P

_mesh = plsc.VectorSubcoreMesh(core_axis_name="c", subcore_axis_name="s")
_cparams = pltpu.CompilerParams(needs_layout_passes=False)


@functools.partial(
    pl.kernel,
    out_type=(
        jax.ShapeDtypeStruct((NW, R), jnp.float32),
        jax.ShapeDtypeStruct((NW, R), jnp.float32),
    ),
    mesh=_mesh,
    compiler_params=_cparams,
    scratch_types=[
        pltpu.VMEM((LANES, R), jnp.float32),
        pltpu.VMEM((R,), jnp.float32),
        pltpu.VMEM((KE,), jnp.int32),
    ],
)
def _k_deg(src_hbm, dst_hbm, outu_hbm, outi_hbm, tab, robuf, sbuf):
  c = lax.axis_index("c")
  s = lax.axis_index("s")
  wid = c * NS + s
  lane = lax.iota(jnp.int32, LANES)
  ones = jnp.ones((LANES,), jnp.float32)
  zeros = jnp.zeros((LANES,), jnp.float32)

  def ztab(r, _):
    for l in range(LANES):
      tab[l, pl.ds(r * LANES, LANES)] = zeros
    return 0

  def count(idx_hbm):
    def body(j, _):
      pltpu.sync_copy(idx_hbm.at[wid, j], sbuf)
      for t in range(KE // LANES):
        idxv = sbuf[pl.ds(t * LANES, LANES)]
        plsc.addupdate_scatter(tab, [lane, idxv], ones)
      return 0
    lax.fori_loop(0, CH_E, body, 0)

  def reduce(r, _):
    acc = tab[0, pl.ds(r * LANES, LANES)]
    for l in range(1, LANES):
      acc = acc + tab[l, pl.ds(r * LANES, LANES)]
    robuf[pl.ds(r * LANES, LANES)] = acc
    return 0

  lax.fori_loop(0, R // LANES, ztab, 0)
  count(src_hbm)
  lax.fori_loop(0, R // LANES, reduce, 0)
  pltpu.sync_copy(robuf, outu_hbm.at[wid])

  lax.fori_loop(0, R // LANES, ztab, 0)
  count(dst_hbm)
  lax.fori_loop(0, R // LANES, reduce, 0)
  pltpu.sync_copy(robuf, outi_hbm.at[wid])


@functools.partial(
    pl.kernel,
    out_type=(
        jax.ShapeDtypeStruct((NC, R, D), jnp.float32),
        jax.ShapeDtypeStruct((NC, R, D), jnp.float32),
    ),
    mesh=_mesh,
    compiler_params=_cparams,
    scratch_types=[
        pltpu.VMEM_SHARED((R, D), jnp.float32),
        pltpu.VMEM_SHARED((R, D), jnp.float32),
        pltpu.VMEM((KE,), jnp.int32),
        pltpu.VMEM((KE,), jnp.int32),
        pltpu.VMEM((KE,), jnp.int32),
        pltpu.VMEM((KE,), jnp.int32),
        pltpu.VMEM((KE, D), jnp.float32),
        pltpu.VMEM((KE, D), jnp.float32),
        pltpu.VMEM((KE, D), jnp.float32),
        pltpu.VMEM((KE, D), jnp.float32),
        pltpu.SemaphoreType.DMA,
        pltpu.SemaphoreType.DMA,
        pltpu.SemaphoreType.DMA,
        pltpu.SemaphoreType.DMA,
        pltpu.SemaphoreType.DMA,
        pltpu.SemaphoreType.DMA,
        pltpu.SemaphoreType.DMA,
        pltpu.SemaphoreType.DMA,
    ],
)
def _k_layer(src_hbm, dst_hbm, pu_hbm, pi_hbm, outu_hbm, outi_hbm,
             accu_sh, acci_sh,
             sbuf0, dbuf0, sbuf1, dbuf1,
             ru0, ri0, ru1, ri1,
             xu0, xi0, xu1, xi1, gu0, gi0, gu1, gi1):
  c = lax.axis_index("c")
  s = lax.axis_index("s")
  wid = c * NS + s
  sets = (
      (sbuf0, dbuf0, ru0, ri0, xu0, xi0, gu0, gi0),
      (sbuf1, dbuf1, ru1, ri1, xu1, xi1, gu1, gi1),
  )
  M = CH_E // 2

  zeros = jnp.zeros((LANES,), jnp.float32)

  def zrow(r, _):
    for cc in range(D // LANES):
      ru0[r, pl.ds(cc * LANES, LANES)] = zeros
    return 0

  lax.fori_loop(0, KE, zrow, 0)
  for base in range(0, RPT, KE):
    pltpu.sync_copy(ru0, accu_sh.at[pl.ds(s * RPT + base, KE)])
    pltpu.sync_copy(ru0, acci_sh.at[pl.ds(s * RPT + base, KE)])

  pltpu.sync_copy(src_hbm.at[wid, 0], sbuf0)
  pltpu.sync_copy(dst_hbm.at[wid, 0], dbuf0)
  pltpu.async_copy(pu_hbm.at[sbuf0], ru0, gu0)
  pltpu.async_copy(pi_hbm.at[dbuf0], ri0, gi0)
  pltpu.async_copy(src_hbm.at[wid, 1], sbuf1, xu1)
  pltpu.async_copy(dst_hbm.at[wid, 1], dbuf1, xi1)
  plsc.subcore_barrier()

  def substep(m, j, cur, nxt, has_next, has_next2):
    csb, cdb, cru, cri, cxu, cxi, cgu, cgi = cur
    nsb, ndb, nru, nri, nxu, nxi, ngu, ngi = nxt

    def issue_next():
      pltpu.make_async_copy(src_hbm.at[wid, j + 1], nsb, nxu).wait()
      pltpu.make_async_copy(dst_hbm.at[wid, j + 1], ndb, nxi).wait()
      pltpu.async_copy(pu_hbm.at[nsb], nru, ngu)
      pltpu.async_copy(pi_hbm.at[ndb], nri, ngi)

    if has_next is True:
      issue_next()
    else:
      pl.when(has_next)(issue_next)

    pltpu.make_async_copy(pu_hbm.at[csb], cru, cgu).wait()
    pltpu.make_async_copy(pi_hbm.at[cdb], cri, cgi).wait()
    pltpu.sync_copy(cru, acci_sh.at[cdb], add=True)
    pltpu.sync_copy(cri, accu_sh.at[csb], add=True)

    def issue_idx2():
      pltpu.async_copy(src_hbm.at[wid, j + 2], csb, cxu)
      pltpu.async_copy(dst_hbm.at[wid, j + 2], cdb, cxi)

    if has_next2 is True:
      issue_idx2()
    else:
      pl.when(has_next2)(issue_idx2)

  def body(m, _):
    not_last = m < M - 1
    substep(m, 2 * m, sets[0], sets[1], True, not_last)
    substep(m, 2 * m + 1, sets[1], sets[0], not_last, not_last)
    return 0

  lax.fori_loop(0, M, body, 0)
  plsc.subcore_barrier()

  for base in range(0, RPT, KE):
    rb = pl.ds(s * RPT + base, KE)
    pltpu.sync_copy(accu_sh.at[rb], ru0)
    pltpu.sync_copy(ru0, outu_hbm.at[c, rb])
    pltpu.sync_copy(acci_sh.at[rb], ri0)
    pltpu.sync_copy(ri0, outi_hbm.at[c, rb])


def _dot_chunk(rows_a, rows_b, out_vm, j):
  lane = lax.iota(jnp.int32, LANES)
  rowis = [lane + g * LANES for g in range(KP // LANES)]

  def dbody(d, tots):
    col = jnp.full((LANES,), d, jnp.int32)
    return tuple(
        t + (plsc.load_gather(rows_a, [rowis[g], col])
             * plsc.load_gather(rows_b, [rowis[g], col]))
        for g, t in enumerate(tots))

  tots = lax.fori_loop(
      0, D, dbody,
      tuple(jnp.zeros((LANES,), jnp.float32) for _ in range(KP // LANES)))
  for g, t in enumerate(tots):
    out_vm[j, pl.ds(g * LANES, LANES)] = t


@functools.partial(
    pl.kernel,
    out_type=jax.ShapeDtypeStruct((NW, CHS, KP), jnp.float32),
    mesh=_mesh,
    compiler_params=_cparams,
    scratch_types=[
        pltpu.VMEM((KP,), jnp.int32),
        pltpu.VMEM((KP,), jnp.int32),
        pltpu.VMEM((KP,), jnp.int32),
        pltpu.VMEM((KP,), jnp.int32),
        pltpu.VMEM((KP, D), jnp.float32),
        pltpu.VMEM((KP, D), jnp.float32),
        pltpu.VMEM((KP, D), jnp.float32),
        pltpu.VMEM((KP, D), jnp.float32),
        pltpu.VMEM((CHS, KP), jnp.float32),
        pltpu.SemaphoreType.DMA,
        pltpu.SemaphoreType.DMA,
        pltpu.SemaphoreType.DMA,
        pltpu.SemaphoreType.DMA,
        pltpu.SemaphoreType.DMA,
        pltpu.SemaphoreType.DMA,
        pltpu.SemaphoreType.DMA,
        pltpu.SemaphoreType.DMA,
    ],
)
def _k_score(ru_hbm, ri_hbm, uidx_hbm, iidx_hbm, out_hbm,
             abuf0, bbuf0, abuf1, bbuf1,
             rows_a0, rows_b0, rows_a1, rows_b1,
             out_vm,
             xa0, xb0, xa1, xb1, ga0, gb0, ga1, gb1):
  c = lax.axis_index("c")
  s = lax.axis_index("s")
  wid = c * NS + s
  sets = (
      (abuf0, bbuf0, rows_a0, rows_b0, xa0, xb0, ga0, gb0),
      (abuf1, bbuf1, rows_a1, rows_b1, xa1, xb1, ga1, gb1),
  )
  M = CHS // 2

  pltpu.sync_copy(uidx_hbm.at[wid, 0], abuf0)
  pltpu.sync_copy(iidx_hbm.at[wid, 0], bbuf0)
  pltpu.async_copy(ru_hbm.at[abuf0], rows_a0, ga0)
  pltpu.async_copy(ri_hbm.at[bbuf0], rows_b0, gb0)
  pltpu.async_copy(uidx_hbm.at[wid, 1], abuf1, xa1)
  pltpu.async_copy(iidx_hbm.at[wid, 1], bbuf1, xb1)

  def substep(j, cur, nxt, has_next, has_next2):
    cab, cbb, cra, crb, cxa, cxb, cga, cgb = cur
    nab, nbb, nra, nrb, nxa, nxb, nga, ngb = nxt

    def issue_next():
      pltpu.make_async_copy(uidx_hbm.at[wid, j + 1], nab, nxa).wait()
      pltpu.make_async_copy(iidx_hbm.at[wid, j + 1], nbb, nxb).wait()
      pltpu.async_copy(ru_hbm.at[nab], nra, nga)
      pltpu.async_copy(ri_hbm.at[nbb], nrb, ngb)

    if has_next is True:
      issue_next()
    else:
      pl.when(has_next)(issue_next)

    pltpu.make_async_copy(ru_hbm.at[cab], cra, cga).wait()
    pltpu.make_async_copy(ri_hbm.at[cbb], crb, cgb).wait()

    def issue_idx2():
      pltpu.async_copy(uidx_hbm.at[wid, j + 2], cab, cxa)
      pltpu.async_copy(iidx_hbm.at[wid, j + 2], cbb, cxb)

    if has_next2 is True:
      issue_idx2()
    else:
      pl.when(has_next2)(issue_idx2)

    _dot_chunk(cra, crb, out_vm, j)

  def body(m, _):
    not_last = m < M - 1
    substep(2 * m, sets[0], sets[1], True, not_last)
    substep(2 * m + 1, sets[1], sets[0], not_last, not_last)
    return 0

  lax.fori_loop(0, M, body, 0)
  pltpu.sync_copy(out_vm, out_hbm.at[wid])


def _pad_reshape_idx(a, total, fill, ch, k):
  pad = total - a.shape[0]
  a = jnp.concatenate([a, jnp.full((pad,), fill, jnp.int32)])
  return a.reshape(NW, ch, k)


def _pad_rows(m):
  return jnp.concatenate(
      [m, jnp.zeros((R - m.shape[0], m.shape[1]), m.dtype)], axis=0)


def kernel(edge_index, pos_edge_index, neg_edge_index, user_emb, item_emb):
  src = edge_index[0]
  dst = edge_index[1]
  src3 = _pad_reshape_idx(src, EPAD, U, CH_E, KE)
  dst3 = _pad_reshape_idx(dst, EPAD, I, CH_E, KE)

  du, di = _k_deg(src3, dst3)
  deg_u = jnp.sum(du, axis=0)[:U]
  deg_i = jnp.sum(di, axis=0)[:I]
  inv_su = lax.rsqrt(jnp.maximum(deg_u, 1.0))
  inv_si = lax.rsqrt(jnp.maximum(deg_i, 1.0))

  h_u, h_i = user_emb, item_emb
  res_u, res_i = user_emb, item_emb
  for l in range(L):
    pu = _pad_rows(h_u * inv_su[:, None])
    pi = _pad_rows(h_i * inv_si[:, None])
    agg_u, agg_i = _k_layer(src3, dst3, pu, pi)
    h_u = (agg_u[0] + agg_u[1])[:U] * inv_su[:, None]
    h_i = (agg_i[0] + agg_i[1])[:I] * inv_si[:, None]
    res_u = res_u + h_u * (1.0 / (l + 2))
    res_i = res_i + h_i * (1.0 / (l + 2))

  pu3 = _pad_reshape_idx(pos_edge_index[0], PPAD, 0, CH_P, KP)
  pi3 = _pad_reshape_idx(pos_edge_index[1], PPAD, 0, CH_P, KP)
  nu3 = _pad_reshape_idx(neg_edge_index[0], PPAD, 0, CH_P, KP)
  ni3 = _pad_reshape_idx(neg_edge_index[1], PPAD, 0, CH_P, KP)
  uidx = jnp.concatenate([pu3, nu3], axis=1)
  iidx = jnp.concatenate([pi3, ni3], axis=1)
  out = _k_score(res_u, res_i, uidx, iidx)
  outp = out[:, :CH_P, :]
  outn = out[:, CH_P:, :]
  pos_score = outp.reshape(-1)[:EP, None]
  neg_score = outn.reshape(-1)[:EP, None]
  return (pos_score, neg_score)


# speedup vs baseline: 6.1759x; 1.2936x over previous
import functools

import jax
import jax.numpy as jnp
from jax import lax
from jax.experimental import pallas as pl
from jax.experimental.pallas import tpu as pltpu
from jax.experimental.pallas import tpu_sc as plsc

U = 5000
I = 5000
E = 320000
EP = 100000
D = 128
L = 3

NC = 2
NS = 16
NW = NC * NS
LANES = 16

KE = 80
KP = 128
R = 5120
RPT = R // NS

EPAD = 327680
CH_E = EPAD // (NW * KE)
PPAD = 102400
CH_P = PPAD // (NW * KP)
CHS = 2 * CH_P

_mesh = plsc.VectorSubcoreMesh(core_axis_name="c", subcore_axis_name="s")
_cparams = pltpu.CompilerParams(needs_layout_passes=False)


@functools.partial(
    pl.kernel,
    out_type=(
        jax.ShapeDtypeStruct((NW, R), jnp.float32),
        jax.ShapeDtypeStruct((NW, R), jnp.float32),
    ),
    mesh=_mesh,
    compiler_params=_cparams,
    scratch_types=[
        pltpu.VMEM((LANES, R), jnp.float32),
        pltpu.VMEM((R,), jnp.float32),
        pltpu.VMEM((KE,), jnp.int32),
        pltpu.VMEM((KE,), jnp.int32),
        pltpu.SemaphoreType.DMA,
        pltpu.SemaphoreType.DMA,
    ],
)
def _k_deg(src_hbm, dst_hbm, outu_hbm, outi_hbm, tab, robuf, sbufa, sbufb,
           xa, xb):
  c = lax.axis_index("c")
  s = lax.axis_index("s")
  wid = c * NS + s
  lane = lax.iota(jnp.int32, LANES)
  ones = jnp.ones((LANES,), jnp.float32)
  zeros = jnp.zeros((LANES,), jnp.float32)

  def ztab(r, _):
    for l in range(LANES):
      tab[l, pl.ds(r * LANES, LANES)] = zeros
    return 0

  def count(idx_hbm):
    pltpu.sync_copy(idx_hbm.at[wid, 0], sbufa)
    pltpu.async_copy(idx_hbm.at[wid, 1], sbufb, xb)

    def scat(buf):
      for t in range(KE // LANES):
        idxv = buf[pl.ds(t * LANES, LANES)]
        plsc.addupdate_scatter(tab, [lane, idxv], ones)

    def body(m, _):
      j = 2 * m

      @pl.when(j + 2 < CH_E)
      def _():
        pltpu.async_copy(idx_hbm.at[wid, j + 2], sbufa, xa)
      scat(sbufa)
      pltpu.make_async_copy(idx_hbm.at[wid, j + 1], sbufb, xb).wait()

      @pl.when(j + 3 < CH_E)
      def _():
        pltpu.async_copy(idx_hbm.at[wid, j + 3], sbufb, xb)
      scat(sbufb)

      @pl.when(j + 2 < CH_E)
      def _():
        pltpu.make_async_copy(idx_hbm.at[wid, j + 2], sbufa, xa).wait()
      return 0

    lax.fori_loop(0, CH_E // 2, body, 0)

  def reduce(r, _):
    acc = tab[0, pl.ds(r * LANES, LANES)]
    for l in range(1, LANES):
      acc = acc + tab[l, pl.ds(r * LANES, LANES)]
    robuf[pl.ds(r * LANES, LANES)] = acc
    return 0

  lax.fori_loop(0, R // LANES, ztab, 0)
  count(src_hbm)
  lax.fori_loop(0, R // LANES, reduce, 0)
  pltpu.sync_copy(robuf, outu_hbm.at[wid])

  lax.fori_loop(0, R // LANES, ztab, 0)
  count(dst_hbm)
  lax.fori_loop(0, R // LANES, reduce, 0)
  pltpu.sync_copy(robuf, outi_hbm.at[wid])


@functools.partial(
    pl.kernel,
    out_type=(
        jax.ShapeDtypeStruct((NC, R, D), jnp.float32),
        jax.ShapeDtypeStruct((NC, R, D), jnp.float32),
    ),
    mesh=_mesh,
    compiler_params=_cparams,
    scratch_types=[
        pltpu.VMEM_SHARED((R, D), jnp.float32),
        pltpu.VMEM_SHARED((R, D), jnp.float32),
        pltpu.VMEM((KE,), jnp.int32),
        pltpu.VMEM((KE,), jnp.int32),
        pltpu.VMEM((KE,), jnp.int32),
        pltpu.VMEM((KE,), jnp.int32),
        pltpu.VMEM((KE, D), jnp.float32),
        pltpu.VMEM((KE, D), jnp.float32),
        pltpu.VMEM((KE, D), jnp.float32),
        pltpu.VMEM((KE, D), jnp.float32),
        pltpu.SemaphoreType.DMA,
        pltpu.SemaphoreType.DMA,
        pltpu.SemaphoreType.DMA,
        pltpu.SemaphoreType.DMA,
        pltpu.SemaphoreType.DMA,
        pltpu.SemaphoreType.DMA,
        pltpu.SemaphoreType.DMA,
        pltpu.SemaphoreType.DMA,
    ],
)
def _k_layer(src_hbm, dst_hbm, pu_hbm, pi_hbm, outu_hbm, outi_hbm,
             accu_sh, acci_sh,
             sbuf0, dbuf0, sbuf1, dbuf1,
             ru0, ri0, ru1, ri1,
             xu0, xi0, xu1, xi1, gu0, gi0, gu1, gi1):
  c = lax.axis_index("c")
  s = lax.axis_index("s")
  wid = c * NS + s
  sets = (
      (sbuf0, dbuf0, ru0, ri0, xu0, xi0, gu0, gi0),
      (sbuf1, dbuf1, ru1, ri1, xu1, xi1, gu1, gi1),
  )
  M = CH_E // 2

  zeros = jnp.zeros((LANES,), jnp.float32)

  def zrow(r, _):
    for cc in range(D // LANES):
      ru0[r, pl.ds(cc * LANES, LANES)] = zeros
    return 0

  lax.fori_loop(0, KE, zrow, 0)
  for base in range(0, RPT, KE):
    pltpu.sync_copy(ru0, accu_sh.at[pl.ds(s * RPT + base, KE)])
    pltpu.sync_copy(ru0, acci_sh.at[pl.ds(s * RPT + base, KE)])

  pltpu.sync_copy(src_hbm.at[wid, 0], sbuf0)
  pltpu.sync_copy(dst_hbm.at[wid, 0], dbuf0)
  pltpu.async_copy(pu_hbm.at[sbuf0], ru0, gu0)
  pltpu.async_copy(pi_hbm.at[dbuf0], ri0, gi0)
  pltpu.async_copy(src_hbm.at[wid, 1], sbuf1, xu1)
  pltpu.async_copy(dst_hbm.at[wid, 1], dbuf1, xi1)
  plsc.subcore_barrier()

  def substep(m, j, cur, nxt, has_next, has_next2):
    csb, cdb, cru, cri, cxu, cxi, cgu, cgi = cur
    nsb, ndb, nru, nri, nxu, nxi, ngu, ngi = nxt

    def issue_next():
      pltpu.make_async_copy(src_hbm.at[wid, j + 1], nsb, nxu).wait()
      pltpu.make_async_copy(dst_hbm.at[wid, j + 1], ndb, nxi).wait()
      pltpu.async_copy(pu_hbm.at[nsb], nru, ngu)
      pltpu.async_copy(pi_hbm.at[ndb], nri, ngi)

    if has_next is True:
      issue_next()
    else:
      pl.when(has_next)(issue_next)

    pltpu.make_async_copy(pu_hbm.at[csb], cru, cgu).wait()
    pltpu.make_async_copy(pi_hbm.at[cdb], cri, cgi).wait()
    pltpu.sync_copy(cru, acci_sh.at[cdb], add=True)
    pltpu.sync_copy(cri, accu_sh.at[csb], add=True)

    def issue_idx2():
      pltpu.async_copy(src_hbm.at[wid, j + 2], csb, cxu)
      pltpu.async_copy(dst_hbm.at[wid, j + 2], cdb, cxi)

    if has_next2 is True:
      issue_idx2()
    else:
      pl.when(has_next2)(issue_idx2)

  def body(m, _):
    not_last = m < M - 1
    substep(m, 2 * m, sets[0], sets[1], True, not_last)
    substep(m, 2 * m + 1, sets[1], sets[0], not_last, not_last)
    return 0

  lax.fori_loop(0, M, body, 0)
  plsc.subcore_barrier()

  for base in range(0, RPT, KE):
    rb = pl.ds(s * RPT + base, KE)
    pltpu.sync_copy(accu_sh.at[rb], ru0)
    pltpu.sync_copy(ru0, outu_hbm.at[c, rb])
    pltpu.sync_copy(acci_sh.at[rb], ri0)
    pltpu.sync_copy(ri0, outi_hbm.at[c, rb])


def _dot_chunk(rows_a, rows_b, scr, out_vm, j):
  lane = lax.iota(jnp.int32, LANES)

  def tbody(t, _):
    for g in range(KP // LANES):
      e = g * LANES + t
      acc = rows_a[e, pl.ds(0, LANES)] * rows_b[e, pl.ds(0, LANES)]
      for cc in range(1, D // LANES):
        acc = acc + (rows_a[e, pl.ds(cc * LANES, LANES)]
                     * rows_b[e, pl.ds(cc * LANES, LANES)])
      scr[e, pl.ds(0, LANES)] = acc
    return 0

  lax.fori_loop(0, LANES, tbody, 0)
  for g in range(KP // LANES):
    rowi = lane + g * LANES
    tot = plsc.load_gather(scr, [rowi, lane])
    for c in range(1, LANES):
      col = jnp.bitwise_and(lane + c, LANES - 1)
      tot = tot + plsc.load_gather(scr, [rowi, col])
    out_vm[j, pl.ds(g * LANES, LANES)] = tot


@functools.partial(
    pl.kernel,
    out_type=jax.ShapeDtypeStruct((NW, CHS, KP), jnp.float32),
    mesh=_mesh,
    compiler_params=_cparams,
    scratch_types=[
        pltpu.VMEM((KP,), jnp.int32),
        pltpu.VMEM((KP,), jnp.int32),
        pltpu.VMEM((KP,), jnp.int32),
        pltpu.VMEM((KP,), jnp.int32),
        pltpu.VMEM((KP, D), jnp.float32),
        pltpu.VMEM((KP, D), jnp.float32),
        pltpu.VMEM((KP, D), jnp.float32),
        pltpu.VMEM((KP, D), jnp.float32),
        pltpu.VMEM((KP, LANES), jnp.float32),
        pltpu.VMEM((CHS, KP), jnp.float32),
        pltpu.SemaphoreType.DMA,
        pltpu.SemaphoreType.DMA,
        pltpu.SemaphoreType.DMA,
        pltpu.SemaphoreType.DMA,
        pltpu.SemaphoreType.DMA,
        pltpu.SemaphoreType.DMA,
        pltpu.SemaphoreType.DMA,
        pltpu.SemaphoreType.DMA,
    ],
)
def _k_score(ru_hbm, ri_hbm, uidx_hbm, iidx_hbm, out_hbm,
             abuf0, bbuf0, abuf1, bbuf1,
             rows_a0, rows_b0, rows_a1, rows_b1,
             scr, out_vm,
             xa0, xb0, xa1, xb1, ga0, gb0, ga1, gb1):
  c = lax.axis_index("c")
  s = lax.axis_index("s")
  wid = c * NS + s
  sets = (
      (abuf0, bbuf0, rows_a0, rows_b0, xa0, xb0, ga0, gb0),
      (abuf1, bbuf1, rows_a1, rows_b1, xa1, xb1, ga1, gb1),
  )
  M = CHS // 2

  pltpu.sync_copy(uidx_hbm.at[wid, 0], abuf0)
  pltpu.sync_copy(iidx_hbm.at[wid, 0], bbuf0)
  pltpu.async_copy(ru_hbm.at[abuf0], rows_a0, ga0)
  pltpu.async_copy(ri_hbm.at[bbuf0], rows_b0, gb0)
  pltpu.async_copy(uidx_hbm.at[wid, 1], abuf1, xa1)
  pltpu.async_copy(iidx_hbm.at[wid, 1], bbuf1, xb1)

  def substep(j, cur, nxt, has_next, has_next2):
    cab, cbb, cra, crb, cxa, cxb, cga, cgb = cur
    nab, nbb, nra, nrb, nxa, nxb, nga, ngb = nxt

    def issue_next():
      pltpu.make_async_copy(uidx_hbm.at[wid, j + 1], nab, nxa).wait()
      pltpu.make_async_copy(iidx_hbm.at[wid, j + 1], nbb, nxb).wait()
      pltpu.async_copy(ru_hbm.at[nab], nra, nga)
      pltpu.async_copy(ri_hbm.at[nbb], nrb, ngb)

    if has_next is True:
      issue_next()
    else:
      pl.when(has_next)(issue_next)

    pltpu.make_async_copy(ru_hbm.at[cab], cra, cga).wait()
    pltpu.make_async_copy(ri_hbm.at[cbb], crb, cgb).wait()

    def issue_idx2():
      pltpu.async_copy(uidx_hbm.at[wid, j + 2], cab, cxa)
      pltpu.async_copy(iidx_hbm.at[wid, j + 2], cbb, cxb)

    if has_next2 is True:
      issue_idx2()
    else:
      pl.when(has_next2)(issue_idx2)

    _dot_chunk(cra, crb, scr, out_vm, j)

  def body(m, _):
    not_last = m < M - 1
    substep(2 * m, sets[0], sets[1], True, not_last)
    substep(2 * m + 1, sets[1], sets[0], not_last, not_last)
    return 0

  lax.fori_loop(0, M, body, 0)
  pltpu.sync_copy(out_vm, out_hbm.at[wid])


def _pad_reshape_idx(a, total, fill, ch, k):
  pad = total - a.shape[0]
  a = jnp.concatenate([a, jnp.full((pad,), fill, jnp.int32)])
  return a.reshape(NW, ch, k)


def _pad_rows(m):
  return jnp.concatenate(
      [m, jnp.zeros((R - m.shape[0], m.shape[1]), m.dtype)], axis=0)


def kernel(edge_index, pos_edge_index, neg_edge_index, user_emb, item_emb):
  src = edge_index[0]
  dst = edge_index[1]
  src3 = _pad_reshape_idx(src, EPAD, U, CH_E, KE)
  dst3 = _pad_reshape_idx(dst, EPAD, I, CH_E, KE)

  du, di = _k_deg(src3, dst3)
  deg_u = jnp.sum(du, axis=0)[:U]
  deg_i = jnp.sum(di, axis=0)[:I]
  inv_su = lax.rsqrt(jnp.maximum(deg_u, 1.0))
  inv_si = lax.rsqrt(jnp.maximum(deg_i, 1.0))

  h_u, h_i = user_emb, item_emb
  res_u, res_i = user_emb, item_emb
  for l in range(L):
    pu = _pad_rows(h_u * inv_su[:, None])
    pi = _pad_rows(h_i * inv_si[:, None])
    agg_u, agg_i = _k_layer(src3, dst3, pu, pi)
    h_u = (agg_u[0] + agg_u[1])[:U] * inv_su[:, None]
    h_i = (agg_i[0] + agg_i[1])[:I] * inv_si[:, None]
    res_u = res_u + h_u * (1.0 / (l + 2))
    res_i = res_i + h_i * (1.0 / (l + 2))

  pu3 = _pad_reshape_idx(pos_edge_index[0], PPAD, 0, CH_P, KP)
  pi3 = _pad_reshape_idx(pos_edge_index[1], PPAD, 0, CH_P, KP)
  nu3 = _pad_reshape_idx(neg_edge_index[0], PPAD, 0, CH_P, KP)
  ni3 = _pad_reshape_idx(neg_edge_index[1], PPAD, 0, CH_P, KP)
  uidx = jnp.concatenate([pu3, nu3], axis=1)
  iidx = jnp.concatenate([pi3, ni3], axis=1)
  out = _k_score(res_u, res_i, uidx, iidx)
  outp = out[:, :CH_P, :]
  outn = out[:, CH_P:, :]
  pos_score = outp.reshape(-1)[:EP, None]
  neg_score = outn.reshape(-1)[:EP, None]
  return (pos_score, neg_score)
